# Initial kernel scaffold; baseline (speedup 1.0000x reference)
import jax
import jax.numpy as jnp
from jax.experimental import pallas as pl


def kernel(p1, idx_i, idx_j, basis, W_pp, b_pp, W_pi, b_pi, W_ii, b_ii):
    raise NotImplementedError("write your pallas kernel here")


# speedup vs baseline: 1.0896x; 1.0896x over previous
import functools

import jax
import jax.numpy as jnp
from jax import lax
from jax.experimental import pallas as pl
from jax.experimental.pallas import tpu as pltpu
from jax.experimental.pallas import tpu_sc as plsc

N = 10000
E = 320000
D = 128
NB = 4

NC = 2
NS = 16
L = 16
NW = NC * NS
EW = E // NW
CH = 40
NCHUNK = EW // CH
NPAD = 10240
RPT = NPAD // NS

_BLK = 1000


def _node_body(p1_ref, wpp_ref, bpp_ref, wpt_i_ref, wpt_j_ref, wii_ref,
               bpi_t_ref, ui_ref, uj_ref):
    h = jnp.tanh(
        jnp.dot(p1_ref[...], wpp_ref[...], preferred_element_type=jnp.float32)
        + bpp_ref[...]
    )
    wii = wii_ref[...]
    for b in range(NB):
        wf_i = jnp.dot(wpt_i_ref[b], wii, preferred_element_type=jnp.float32)
        wf_j = jnp.dot(wpt_j_ref[b], wii, preferred_element_type=jnp.float32)
        bp_b = jnp.dot(bpi_t_ref[b:b + 1, :], wii,
                       preferred_element_type=jnp.float32)
        ui_ref[:, b * D:(b + 1) * D] = (
            jnp.dot(h, wf_i, preferred_element_type=jnp.float32) + bp_b
        )
        uj_ref[:, b * D:(b + 1) * D] = jnp.dot(
            h, wf_j, preferred_element_type=jnp.float32
        )


def _node_call(p1, w_pp, b_pp, wpt_i, wpt_j, w_ii, bpi_t):
    grid = N // _BLK
    return pl.pallas_call(
        _node_body,
        grid=(grid,),
        in_specs=[
            pl.BlockSpec((_BLK, D), lambda i: (i, 0)),
            pl.BlockSpec((D, D), lambda i: (0, 0)),
            pl.BlockSpec((1, D), lambda i: (0, 0)),
            pl.BlockSpec((NB, D, D), lambda i: (0, 0, 0)),
            pl.BlockSpec((NB, D, D), lambda i: (0, 0, 0)),
            pl.BlockSpec((D, D), lambda i: (0, 0)),
            pl.BlockSpec((NB, D), lambda i: (0, 0)),
        ],
        out_specs=[
            pl.BlockSpec((_BLK, NB * D), lambda i: (i, 0)),
            pl.BlockSpec((_BLK, NB * D), lambda i: (i, 0)),
        ],
        out_shape=[
            jax.ShapeDtypeStruct((N, NB * D), jnp.float32),
            jax.ShapeDtypeStruct((N, NB * D), jnp.float32),
        ],
    )(p1, w_pp, b_pp, wpt_i, wpt_j, w_ii, bpi_t)


def _edge_body(ui, uj, idxi, idxj, basis8, bii, zeros_hbm, out,
               idxi_v, idxj_v, basis_v, gi_v, gj_v, y_v, bii_v, acc, sem):
    cid = lax.axis_index("c")
    sid = lax.axis_index("s")
    w = sid * NC + cid

    pltpu.sync_copy(zeros_hbm.at[pl.ds(sid * RPT, RPT)],
                    acc.at[pl.ds(sid * RPT, RPT)])
    pltpu.sync_copy(bii, bii_v)
    plsc.subcore_barrier()

    base0 = w * EW

    def chunk_body(i, _):
        base = base0 + i * CH
        pltpu.sync_copy(idxi.at[pl.ds(base, CH)], idxi_v)
        pltpu.sync_copy(idxj.at[pl.ds(base, CH)], idxj_v)
        pltpu.sync_copy(basis8.at[pl.ds(base * 8, CH * 8)],
                        basis_v.at[pl.ds(0, CH * 8)])
        pltpu.async_copy(ui.at[idxi_v], gi_v, sem).wait()
        pltpu.async_copy(uj.at[idxj_v], gj_v, sem).wait()

        def edge_body(e, _):
            bvec = basis_v[pl.ds(e * 8, L)]
            betas = [
                jnp.take(bvec, jnp.full((L,), b, jnp.int32), mode="fill")
                for b in range(NB)
            ]
            for k8 in range(D // L):
                acc_v = bii_v[pl.ds(k8 * L, L)]
                for b in range(NB):
                    s = (gi_v[e, pl.ds(b * D + k8 * L, L)]
                         + gj_v[e, pl.ds(b * D + k8 * L, L)])
                    acc_v = acc_v + s * betas[b]
                t = 2.0 / (jnp.exp(acc_v * -2.0) + 1.0) - 1.0
                y_v[e, pl.ds(k8 * L, L)] = t
            return ()

        lax.fori_loop(0, CH, edge_body, (), unroll=False)
        pltpu.sync_copy(y_v, acc.at[idxj_v], add=True)
        return ()

    lax.fori_loop(0, NCHUNK, chunk_body, (), unroll=False)
    plsc.subcore_barrier()
    pltpu.sync_copy(acc.at[pl.ds(sid * RPT, RPT)],
                    out.at[pl.ds(cid * NPAD + sid * RPT, RPT)])


def _edge_call(ui, uj, idx_i, idx_j, basis8, b_ii, zeros):
    mesh = plsc.VectorSubcoreMesh(
        core_axis_name="c", subcore_axis_name="s",
        num_cores=NC, num_subcores=NS,
    )
    f = functools.partial(
        pl.kernel,
        out_type=jax.ShapeDtypeStruct((NC * NPAD, D), jnp.float32),
        mesh=mesh,
        scratch_types=[
            pltpu.VMEM((CH,), jnp.int32),
            pltpu.VMEM((CH,), jnp.int32),
            pltpu.VMEM((CH * 8 + L,), jnp.float32),
            pltpu.VMEM((CH, NB * D), jnp.float32),
            pltpu.VMEM((CH, NB * D), jnp.float32),
            pltpu.VMEM((CH, D), jnp.float32),
            pltpu.VMEM((D,), jnp.float32),
            pltpu.VMEM_SHARED((NPAD, D), jnp.float32),
            pltpu.SemaphoreType.DMA,
        ],
    )(_edge_body)
    return f(ui, uj, idx_i, idx_j, basis8, b_ii, zeros)


def _combine_body(pa_ref, pb_ref, o_ref):
    o_ref[...] = pa_ref[...] + pb_ref[...]


def _combine_call(partials):
    blk = 80
    grid = N // blk
    return pl.pallas_call(
        _combine_body,
        grid=(grid,),
        in_specs=[
            pl.BlockSpec((blk, D), lambda i: (i, 0)),
            pl.BlockSpec((blk, D), lambda i: (i + NPAD // 80, 0)),
        ],
        out_specs=pl.BlockSpec((blk, D), lambda i: (i, 0)),
        out_shape=jax.ShapeDtypeStruct((N, D), jnp.float32),
    )(partials, partials)


def kernel(p1, idx_i, idx_j, basis, W_pp, b_pp, W_pi, b_pi, W_ii, b_ii):
    idx_i = idx_i.astype(jnp.int32)
    idx_j = idx_j.astype(jnp.int32)
    wpt_i = W_pi[:D].reshape(D, D, NB).transpose(2, 0, 1)
    wpt_j = W_pi[D:].reshape(D, D, NB).transpose(2, 0, 1)
    bpi_t = b_pi.reshape(D, NB).T
    basis8 = jnp.pad(basis, ((0, 0), (0, 8 - NB))).reshape(-1)
    zeros = jnp.zeros((NPAD, D), jnp.float32)

    ui, uj = _node_call(p1, W_pp, b_pp.reshape(1, D), wpt_i, wpt_j, W_ii,
                        bpi_t)
    partials = _edge_call(ui, uj, idx_i, idx_j, basis8, b_ii, zeros)
    return _combine_call(partials)


# speedup vs baseline: 1.3797x; 1.2663x over previous
import functools

import jax
import jax.numpy as jnp
from jax import lax
from jax.experimental import pallas as pl
from jax.experimental.pallas import tpu as pltpu
from jax.experimental.pallas import tpu_sc as plsc

N = 10000
E = 320000
D = 128
NB = 4

NC = 2
NS = 16
L = 16
NW = NC * NS
EW = E // NW
CH = 16
NCHUNK = EW // CH
NPAD = 10240
RPT = NPAD // NS

_BLK = 1000


def _node_body(p1_ref, wpp_ref, bpp_ref, wpt_i_ref, wpt_j_ref, wii_ref,
               bpi_t_ref, ui_ref, uj_ref):
    h = jnp.tanh(
        jnp.dot(p1_ref[...], wpp_ref[...], preferred_element_type=jnp.float32)
        + bpp_ref[...]
    )
    wii = wii_ref[...]
    for b in range(NB):
        wf_i = jnp.dot(wpt_i_ref[b], wii, preferred_element_type=jnp.float32)
        wf_j = jnp.dot(wpt_j_ref[b], wii, preferred_element_type=jnp.float32)
        bp_b = jnp.dot(bpi_t_ref[b:b + 1, :], wii,
                       preferred_element_type=jnp.float32)
        ui_ref[:, b * D:(b + 1) * D] = (
            jnp.dot(h, wf_i, preferred_element_type=jnp.float32) + bp_b
        )
        uj_ref[:, b * D:(b + 1) * D] = jnp.dot(
            h, wf_j, preferred_element_type=jnp.float32
        )


def _node_call(p1, w_pp, b_pp, wpt_i, wpt_j, w_ii, bpi_t):
    grid = N // _BLK
    return pl.pallas_call(
        _node_body,
        grid=(grid,),
        in_specs=[
            pl.BlockSpec((_BLK, D), lambda i: (i, 0)),
            pl.BlockSpec((D, D), lambda i: (0, 0)),
            pl.BlockSpec((1, D), lambda i: (0, 0)),
            pl.BlockSpec((NB, D, D), lambda i: (0, 0, 0)),
            pl.BlockSpec((NB, D, D), lambda i: (0, 0, 0)),
            pl.BlockSpec((D, D), lambda i: (0, 0)),
            pl.BlockSpec((NB, D), lambda i: (0, 0)),
        ],
        out_specs=[
            pl.BlockSpec((_BLK, NB * D), lambda i: (i, 0)),
            pl.BlockSpec((_BLK, NB * D), lambda i: (i, 0)),
        ],
        out_shape=[
            jax.ShapeDtypeStruct((N, NB * D), jnp.float32),
            jax.ShapeDtypeStruct((N, NB * D), jnp.float32),
        ],
    )(p1, w_pp, b_pp, wpt_i, wpt_j, w_ii, bpi_t)


def _edge_body(ui, uj, idxi, idxj, basis8, bii, zeros_hbm, out,
               idxi_v0, idxi_v1, idxj_v0, idxj_v1, sj_v0, sj_v1,
               basis_v0, basis_v1, gi_v0, gi_v1, gj_v0, gj_v1,
               y_v0, y_v1, bii_v, acc,
               sem_ib0, sem_ib1, sem_g0, sem_g1, sem_s0, sem_s1):
    cid = lax.axis_index("c")
    sid = lax.axis_index("s")
    w = sid * NC + cid

    idxi_v = (idxi_v0, idxi_v1)
    idxj_v = (idxj_v0, idxj_v1)
    sj_v = (sj_v0, sj_v1)
    basis_v = (basis_v0, basis_v1)
    gi_v = (gi_v0, gi_v1)
    gj_v = (gj_v0, gj_v1)
    y_v = (y_v0, y_v1)
    sem_ib = (sem_ib0, sem_ib1)
    sem_g = (sem_g0, sem_g1)
    sem_s = (sem_s0, sem_s1)

    pltpu.sync_copy(zeros_hbm.at[pl.ds(sid * RPT, RPT)],
                    acc.at[pl.ds(sid * RPT, RPT)])
    pltpu.sync_copy(bii, bii_v)
    plsc.subcore_barrier()

    base0 = w * EW

    def issue_ib(c, p):
        base = base0 + c * CH
        pltpu.async_copy(idxi.at[pl.ds(base, CH)], idxi_v[p], sem_ib[p])
        pltpu.async_copy(idxj.at[pl.ds(base, CH)], idxj_v[p], sem_ib[p])
        pltpu.async_copy(basis8.at[pl.ds(base * 8, CH * 8)],
                         basis_v[p].at[pl.ds(0, CH * 8)], sem_ib[p])

    def wait_ib(p):
        pltpu.make_async_copy(idxi.at[pl.ds(0, CH)], idxi_v[p],
                              sem_ib[p]).wait()
        pltpu.make_async_copy(idxj.at[pl.ds(0, CH)], idxj_v[p],
                              sem_ib[p]).wait()
        pltpu.make_async_copy(basis8.at[pl.ds(0, CH * 8)],
                              basis_v[p].at[pl.ds(0, CH * 8)],
                              sem_ib[p]).wait()

    def issue_g(p):
        pltpu.async_copy(ui.at[idxi_v[p]], gi_v[p], sem_g[p])
        pltpu.async_copy(uj.at[idxj_v[p]], gj_v[p], sem_g[p])

    def wait_g(p):
        pltpu.make_async_copy(ui.at[idxi_v[p]], gi_v[p], sem_g[p]).wait()
        pltpu.make_async_copy(uj.at[idxj_v[p]], gj_v[p], sem_g[p]).wait()

    def issue_s(p):
        pltpu.async_copy(y_v[p], acc.at[sj_v[p]], sem_s[p], add=True)

    def wait_s(p):
        pltpu.make_async_copy(y_v[p], acc.at[sj_v[p]], sem_s[p]).wait()

    def compute(p):
        def edge_body(e, _):
            bvec = basis_v[p][pl.ds(e * 8, L)]
            betas = [
                jnp.take(bvec, jnp.full((L,), b, jnp.int32), mode="fill")
                for b in range(NB)
            ]
            for k8 in range(D // L):
                acc_v = bii_v[pl.ds(k8 * L, L)]
                for b in range(NB):
                    s = (gi_v[p][e, pl.ds(b * D + k8 * L, L)]
                         + gj_v[p][e, pl.ds(b * D + k8 * L, L)])
                    acc_v = acc_v + s * betas[b]
                t = 2.0 / (jnp.exp(acc_v * -2.0) + 1.0) - 1.0
                y_v[p][e, pl.ds(k8 * L, L)] = t
            return ()

        lax.fori_loop(0, CH, edge_body, (), unroll=False)

    def chunk(c, p):
        wait_g(p)
        def _next_gather():
            wait_ib(1 - p)
            issue_g(1 - p)
        pl.when(c + 1 < NCHUNK)(_next_gather)
        pl.when(c >= 2)(lambda: wait_s(p))
        sj_v[p][...] = idxj_v[p][...]
        compute(p)
        issue_s(p)
        pl.when(c + 2 < NCHUNK)(lambda: issue_ib(c + 2, p))

    issue_ib(0, 0)
    issue_ib(1, 1)
    wait_ib(0)
    issue_g(0)

    def pair_body(q, _):
        chunk(q * 2, 0)
        chunk(q * 2 + 1, 1)
        return ()

    lax.fori_loop(0, NCHUNK // 2, pair_body, (), unroll=False)
    if NCHUNK % 2:
        chunk(NCHUNK - 1, 0)
    wait_s((NCHUNK - 1) % 2)
    wait_s(NCHUNK % 2)

    plsc.subcore_barrier()
    pltpu.sync_copy(acc.at[pl.ds(sid * RPT, RPT)],
                    out.at[pl.ds(cid * NPAD + sid * RPT, RPT)])


def _edge_call(ui, uj, idx_i, idx_j, basis8, b_ii, zeros):
    mesh = plsc.VectorSubcoreMesh(
        core_axis_name="c", subcore_axis_name="s",
        num_cores=NC, num_subcores=NS,
    )
    f = functools.partial(
        pl.kernel,
        out_type=jax.ShapeDtypeStruct((NC * NPAD, D), jnp.float32),
        mesh=mesh,
        scratch_types=[
            pltpu.VMEM((CH,), jnp.int32),
            pltpu.VMEM((CH,), jnp.int32),
            pltpu.VMEM((CH,), jnp.int32),
            pltpu.VMEM((CH,), jnp.int32),
            pltpu.VMEM((CH,), jnp.int32),
            pltpu.VMEM((CH,), jnp.int32),
            pltpu.VMEM((CH * 8 + L,), jnp.float32),
            pltpu.VMEM((CH * 8 + L,), jnp.float32),
            pltpu.VMEM((CH, NB * D), jnp.float32),
            pltpu.VMEM((CH, NB * D), jnp.float32),
            pltpu.VMEM((CH, NB * D), jnp.float32),
            pltpu.VMEM((CH, NB * D), jnp.float32),
            pltpu.VMEM((CH, D), jnp.float32),
            pltpu.VMEM((CH, D), jnp.float32),
            pltpu.VMEM((D,), jnp.float32),
            pltpu.VMEM_SHARED((NPAD, D), jnp.float32),
            pltpu.SemaphoreType.DMA,
            pltpu.SemaphoreType.DMA,
            pltpu.SemaphoreType.DMA,
            pltpu.SemaphoreType.DMA,
            pltpu.SemaphoreType.DMA,
            pltpu.SemaphoreType.DMA,
        ],
    )(_edge_body)
    return f(ui, uj, idx_i, idx_j, basis8, b_ii, zeros)


def _combine_body(pa_ref, pb_ref, o_ref):
    o_ref[...] = pa_ref[...] + pb_ref[...]


def _combine_call(partials):
    blk = 80
    grid = N // blk
    return pl.pallas_call(
        _combine_body,
        grid=(grid,),
        in_specs=[
            pl.BlockSpec((blk, D), lambda i: (i, 0)),
            pl.BlockSpec((blk, D), lambda i: (i + NPAD // 80, 0)),
        ],
        out_specs=pl.BlockSpec((blk, D), lambda i: (i, 0)),
        out_shape=jax.ShapeDtypeStruct((N, D), jnp.float32),
    )(partials, partials)


def kernel(p1, idx_i, idx_j, basis, W_pp, b_pp, W_pi, b_pi, W_ii, b_ii):
    idx_i = idx_i.astype(jnp.int32)
    idx_j = idx_j.astype(jnp.int32)
    wpt_i = W_pi[:D].reshape(D, D, NB).transpose(2, 0, 1)
    wpt_j = W_pi[D:].reshape(D, D, NB).transpose(2, 0, 1)
    bpi_t = b_pi.reshape(D, NB).T
    basis8 = jnp.pad(basis, ((0, 0), (0, 8 - NB))).reshape(-1)
    zeros = jnp.zeros((NPAD, D), jnp.float32)

    ui, uj = _node_call(p1, W_pp, b_pp.reshape(1, D), wpt_i, wpt_j, W_ii,
                        bpi_t)
    partials = _edge_call(ui, uj, idx_i, idx_j, basis8, b_ii, zeros)
    return _combine_call(partials)
